# baseline (speedup 1.0000x reference)
import jax
import jax.numpy as jnp
from jax.experimental import pallas as pl
from jax.experimental.pallas import tpu as pltpu

EPS = 1e-8
_BB = 16
_RC = 128
_NH = 2


def _body(key_ref, beta_ref, mode_ref, w_ref, mem_ref, link_ref,
          read_ref, wout_ref, wacc_ref):
    NH_ROWS = link_ref.shape[2]
    N = link_ref.shape[3]
    W = key_ref.shape[2]
    h = pl.program_id(1)
    ones_w = jnp.ones((1, W), dtype=jnp.bfloat16)

    def link_fb(off):
        fb = []
        for i in range(_BB):
            w_bf = w_ref[i].astype(jnp.bfloat16)
            f_parts = []
            b = None
            for r in range(0, NH_ROWS, _RC):
                chunk = link_ref[i, 0, r:r + _RC, :].astype(jnp.bfloat16)
                f_parts.append(jax.lax.dot_general(
                    w_bf, chunk, (((1,), (1,)), ((), ())),
                    preferred_element_type=jnp.float32))
                bp = jax.lax.dot_general(
                    w_bf[:, off + r:off + r + _RC], chunk,
                    (((1,), (0,)), ((), ())),
                    preferred_element_type=jnp.float32)
                b = bp if b is None else b + bp
            fb.append((jnp.concatenate(f_parts, axis=1), b))
        return fb

    def probs_of(i):
        mode = mode_ref[i]
        mmax = jnp.max(mode, axis=1, keepdims=True)
        me = jnp.exp(mode - mmax)
        return me / jnp.sum(me, axis=1, keepdims=True)

    @pl.when(h == 0)
    def _():
        fb = link_fb(0)
        for i in range(_BB):
            mem_bf = mem_ref[i].astype(jnp.bfloat16)
            key = key_ref[i]
            probs = probs_of(i)
            beta = 1.0 + jax.nn.softplus(beta_ref[i])

            k = (key / (jnp.abs(key) + EPS)).astype(jnp.bfloat16)
            sim = jax.lax.dot_general(
                k, mem_bf, (((1,), (1,)), ((), ())),
                preferred_element_type=jnp.float32)
            nsq = jax.lax.dot_general(
                ones_w, mem_bf * mem_bf, (((1,), (1,)), ((), ())),
                preferred_element_type=jnp.float32)
            logits = sim / (jnp.sqrt(nsq) + EPS) * beta
            lmax = jnp.max(logits, axis=1, keepdims=True)
            le = jnp.exp(logits - lmax)
            c = le / jnp.sum(le, axis=1, keepdims=True)

            f0, b0 = fb[i]
            acc = probs[:, 1:2] * c + probs[:, 0:1] * b0
            pad = jnp.zeros((1, N - NH_ROWS), dtype=jnp.float32)
            wacc_ref[i] = acc + jnp.concatenate(
                [probs[:, 2:3] * f0, pad], axis=1)

    @pl.when(h == _NH - 1)
    def _():
        off = NH_ROWS * (_NH - 1)
        fb = link_fb(off)
        for i in range(_BB):
            probs = probs_of(i)
            f1, b1 = fb[i]
            pad = jnp.zeros((1, off), dtype=jnp.float32)
            weights = (wacc_ref[i] + probs[:, 0:1] * b1
                       + jnp.concatenate([pad, probs[:, 2:3] * f1], axis=1))
            mem_bf = mem_ref[i].astype(jnp.bfloat16)
            read = jax.lax.dot_general(
                weights.astype(jnp.bfloat16), mem_bf,
                (((1,), (0,)), ((), ())),
                preferred_element_type=jnp.float32)
            read_ref[i] = read
            wout_ref[i] = weights


def kernel(r_key, r_beta, r_mode, r_weights, memory, link_matrix):
    B, N, W = memory.shape
    grid = (B // _BB, _NH)

    key3 = r_key.reshape(B, 1, W)
    beta3 = r_beta.reshape(B, 1, 1)
    mode3 = r_mode.reshape(B, 1, 3)
    w3 = r_weights.reshape(B, 1, N)
    link4 = link_matrix.reshape(B, _NH, N // _NH, N)

    read3, weights3 = pl.pallas_call(
        _body,
        grid=grid,
        in_specs=[
            pl.BlockSpec((_BB, 1, W), lambda i, h: (i, 0, 0)),
            pl.BlockSpec((_BB, 1, 1), lambda i, h: (i, 0, 0)),
            pl.BlockSpec((_BB, 1, 3), lambda i, h: (i, 0, 0)),
            pl.BlockSpec((_BB, 1, N), lambda i, h: (i, 0, 0)),
            pl.BlockSpec((_BB, N, W), lambda i, h: (i, 0, 0)),
            pl.BlockSpec((_BB, 1, N // _NH, N), lambda i, h: (i, h, 0, 0)),
        ],
        out_specs=[
            pl.BlockSpec((_BB, 1, W), lambda i, h: (i, 0, 0)),
            pl.BlockSpec((_BB, 1, N), lambda i, h: (i, 0, 0)),
        ],
        out_shape=[
            jax.ShapeDtypeStruct((B, 1, W), jnp.float32),
            jax.ShapeDtypeStruct((B, 1, N), jnp.float32),
        ],
        scratch_shapes=[pltpu.VMEM((_BB, 1, N), jnp.float32)],
        compiler_params=pltpu.CompilerParams(
            dimension_semantics=("arbitrary", "arbitrary"),
            vmem_limit_bytes=56 * 1024 * 1024,
        ),
        name="dnc_read_head",
    )(key3, beta3, mode3, w3, memory, link4)

    return read3, weights3.reshape(B, N)


# speedup vs baseline: 1.0951x; 1.0951x over previous
import jax
import jax.numpy as jnp
from jax.experimental import pallas as pl
from jax.experimental.pallas import tpu as pltpu

EPS = 1e-8
_BB = 16
_RC = 128


def _body(key_ref, beta_ref, mode_ref, w_ref, mem_ref, link_ref,
          read_ref, wout_ref):
    N = link_ref.shape[1]
    W = key_ref.shape[2]
    ones_w = jnp.ones((1, W), dtype=jnp.bfloat16)

    c_all, probs_all, w_bf_all = [], [], []
    for i in range(_BB):
        mem_bf = mem_ref[i].astype(jnp.bfloat16)
        key = key_ref[i]
        mode = mode_ref[i]

        mmax = jnp.max(mode, axis=1, keepdims=True)
        me = jnp.exp(mode - mmax)
        probs_all.append(me / jnp.sum(me, axis=1, keepdims=True))
        w_bf_all.append(w_ref[i].astype(jnp.bfloat16))

        beta = 1.0 + jax.nn.softplus(beta_ref[i])

        k = (key / (jnp.abs(key) + EPS)).astype(jnp.bfloat16)
        sim = jax.lax.dot_general(
            k, mem_bf, (((1,), (1,)), ((), ())),
            preferred_element_type=jnp.float32)
        nsq = jax.lax.dot_general(
            ones_w, mem_bf * mem_bf, (((1,), (1,)), ((), ())),
            preferred_element_type=jnp.float32)
        logits = sim / (jnp.sqrt(nsq) + EPS) * beta
        lmax = jnp.max(logits, axis=1, keepdims=True)
        le = jnp.exp(logits - lmax)
        c_all.append(le / jnp.sum(le, axis=1, keepdims=True))

    f_parts = [[] for _ in range(_BB)]
    b_acc = [None] * _BB
    for r in range(0, N, _RC):
        for i in range(_BB):
            w_bf = w_bf_all[i]
            chunk = link_ref[i, r:r + _RC, :].astype(jnp.bfloat16)
            f_parts[i].append(jax.lax.dot_general(
                w_bf, chunk, (((1,), (1,)), ((), ())),
                preferred_element_type=jnp.float32))
            b_part = jax.lax.dot_general(
                w_bf[:, r:r + _RC], chunk, (((1,), (0,)), ((), ())),
                preferred_element_type=jnp.float32)
            b_acc[i] = b_part if b_acc[i] is None else b_acc[i] + b_part
    weights_all = []
    for i in range(_BB):
        f = jnp.concatenate(f_parts[i], axis=1)
        probs = probs_all[i]
        weights_all.append(probs[:, 0:1] * b_acc[i] + probs[:, 1:2] * c_all[i]
                           + probs[:, 2:3] * f)

    for i in range(_BB):
        weights = weights_all[i]
        read = jax.lax.dot_general(
            weights.astype(jnp.bfloat16), mem_ref[i].astype(jnp.bfloat16),
            (((1,), (0,)), ((), ())),
            preferred_element_type=jnp.float32)
        read_ref[i] = read
        wout_ref[i] = weights


def kernel(r_key, r_beta, r_mode, r_weights, memory, link_matrix):
    B, N, W = memory.shape
    grid = (B // _BB,)

    key3 = r_key.reshape(B, 1, W)
    beta3 = r_beta.reshape(B, 1, 1)
    mode3 = r_mode.reshape(B, 1, 3)
    w3 = r_weights.reshape(B, 1, N)

    read3, weights3 = pl.pallas_call(
        _body,
        grid=grid,
        in_specs=[
            pl.BlockSpec((_BB, 1, W), lambda i: (i, 0, 0)),
            pl.BlockSpec((_BB, 1, 1), lambda i: (i, 0, 0)),
            pl.BlockSpec((_BB, 1, 3), lambda i: (i, 0, 0)),
            pl.BlockSpec((_BB, 1, N), lambda i: (i, 0, 0)),
            pl.BlockSpec((_BB, N, W), lambda i: (i, 0, 0)),
            pl.BlockSpec((_BB, N, N), lambda i: (i, 0, 0)),
        ],
        out_specs=[
            pl.BlockSpec((_BB, 1, W), lambda i: (i, 0, 0)),
            pl.BlockSpec((_BB, 1, N), lambda i: (i, 0, 0)),
        ],
        out_shape=[
            jax.ShapeDtypeStruct((B, 1, W), jnp.float32),
            jax.ShapeDtypeStruct((B, 1, N), jnp.float32),
        ],
        compiler_params=pltpu.CompilerParams(
            dimension_semantics=("arbitrary",),
            vmem_limit_bytes=56 * 1024 * 1024,
        ),
        name="dnc_read_head",
    )(key3, beta3, mode3, w3, memory, link_matrix)

    return read3, weights3.reshape(B, N)
